# initial kernel scaffold (unmeasured)
import functools

import jax
import jax.numpy as jnp
from jax import lax
from jax.experimental import pallas as pl
from jax.experimental.pallas import tpu as pltpu

N_DEV = 4
M, N = 4096, 2048
CHUNK = M // N_DEV


def _silu(y):
    return y * (1.0 / (1.0 + jnp.exp(-y)))


def kernel(x, w_mat):
    partial = jnp.dot(x, w_mat, preferred_element_type=jnp.float32)

    def body(p_ref, out_ref, acc_ref, local_ref, comm_ref,
             copy_sems, send_sems, recv_sems):
        my = lax.axis_index("i")
        left = jnp.mod(my - 1, N_DEV)
        right = jnp.mod(my + 1, N_DEV)

        barrier = pltpu.get_barrier_semaphore()
        for nbr in (left, right):
            pl.semaphore_signal(
                barrier, inc=1,
                device_id=(nbr,), device_id_type=pl.DeviceIdType.MESH,
            )
        pl.semaphore_wait(barrier, 2)

        cp = pltpu.make_async_copy(
            p_ref.at[pl.ds(my * CHUNK, CHUNK), :], acc_ref, copy_sems.at[0])
        cp.start()
        cp.wait()

        for s in range(N_DEV - 1):
            slot = s % 2
            rdma = pltpu.make_async_remote_copy(
                src_ref=acc_ref,
                dst_ref=comm_ref.at[slot],
                send_sem=send_sems.at[s],
                recv_sem=recv_sems.at[s],
                device_id=(right,),
                device_id_type=pl.DeviceIdType.MESH,
            )
            rdma.start()
            idx = jnp.mod(my - s - 1, N_DEV)
            lcp = pltpu.make_async_copy(
                p_ref.at[pl.ds(idx * CHUNK, CHUNK), :], local_ref,
                copy_sems.at[1])
            lcp.start()
            rdma.wait()
            lcp.wait()
            acc_ref[...] = comm_ref[slot] + local_ref[...]

        acc_ref[...] = _silu(acc_ref[...])
        r = jnp.mod(my + 1, N_DEV)
        ocp = pltpu.make_async_copy(
            acc_ref, out_ref.at[pl.ds(r * CHUNK, CHUNK), :], copy_sems.at[2])
        ocp.start()
        ocp.wait()

        for s in range(N_DEV - 1):
            slot = s % 2
            src = acc_ref if s == 0 else comm_ref.at[(s - 1) % 2]
            rdma = pltpu.make_async_remote_copy(
                src_ref=src,
                dst_ref=comm_ref.at[slot],
                send_sem=send_sems.at[3 + s],
                recv_sem=recv_sems.at[3 + s],
                device_id=(right,),
                device_id_type=pl.DeviceIdType.MESH,
            )
            rdma.start()
            rdma.wait()
            idx = jnp.mod(my - s, N_DEV)
            ocp = pltpu.make_async_copy(
                comm_ref.at[slot], out_ref.at[pl.ds(idx * CHUNK, CHUNK), :],
                copy_sems.at[3])
            ocp.start()
            ocp.wait()

        @functools.partial(
            pl.run_scoped, second_barrier=pltpu.SemaphoreType.REGULAR)
        def _(second_barrier):
            for nbr in (left, right):
                pl.semaphore_signal(
                    second_barrier, inc=1,
                    device_id=(nbr,), device_id_type=pl.DeviceIdType.MESH,
                )
            pl.semaphore_wait(second_barrier, 2)

    return pl.pallas_call(
        body,
        out_shape=jax.ShapeDtypeStruct((M, N), jnp.float32),
        in_specs=[pl.BlockSpec(memory_space=pltpu.ANY)],
        out_specs=pl.BlockSpec(memory_space=pltpu.ANY),
        scratch_shapes=[
            pltpu.VMEM((CHUNK, N), jnp.float32),
            pltpu.VMEM((CHUNK, N), jnp.float32),
            pltpu.VMEM((2, CHUNK, N), jnp.float32),
            pltpu.SemaphoreType.DMA((4,)),
            pltpu.SemaphoreType.DMA((6,)),
            pltpu.SemaphoreType.DMA((6,)),
        ],
        compiler_params=pltpu.CompilerParams(collective_id=0),
    )(partial)


# baseline (device time: 623269 ns/iter reference)
import functools

import jax
import jax.numpy as jnp
from jax import lax
from jax.experimental import pallas as pl
from jax.experimental.pallas import tpu as pltpu

N_DEV = 4
M, N = 4096, 2048
CHUNK = M // N_DEV


def _silu(y):
    return y * (1.0 / (1.0 + jnp.exp(-y)))


def kernel(x, w_mat):
    partial = jnp.dot(x, w_mat, preferred_element_type=jnp.float32)

    def body(p_ref, out_ref, acc_ref, local_ref, comm_ref,
             copy_sems, send_sems, recv_sems):
        my = lax.axis_index("i")
        left = jnp.mod(my - 1, N_DEV)
        right = jnp.mod(my + 1, N_DEV)

        barrier = pltpu.get_barrier_semaphore()
        for nbr in (left, right):
            pl.semaphore_signal(
                barrier, inc=1,
                device_id=(nbr,), device_id_type=pl.DeviceIdType.MESH,
            )
        pl.semaphore_wait(barrier, 2)

        cp = pltpu.make_async_copy(
            p_ref.at[pl.ds(my * CHUNK, CHUNK), :], acc_ref, copy_sems.at[0])
        cp.start()
        cp.wait()

        for s in range(N_DEV - 1):
            slot = s % 2
            rdma = pltpu.make_async_remote_copy(
                src_ref=acc_ref,
                dst_ref=comm_ref.at[slot],
                send_sem=send_sems.at[s],
                recv_sem=recv_sems.at[s],
                device_id=(right,),
                device_id_type=pl.DeviceIdType.MESH,
            )
            rdma.start()
            idx = jnp.mod(my - s - 1, N_DEV)
            lcp = pltpu.make_async_copy(
                p_ref.at[pl.ds(idx * CHUNK, CHUNK), :], local_ref,
                copy_sems.at[1])
            lcp.start()
            rdma.wait()
            lcp.wait()
            acc_ref[...] = comm_ref[slot] + local_ref[...]

        acc_ref[...] = _silu(acc_ref[...])
        r = jnp.mod(my + 1, N_DEV)
        ocp = pltpu.make_async_copy(
            acc_ref, out_ref.at[pl.ds(r * CHUNK, CHUNK), :], copy_sems.at[2])
        ocp.start()
        ocp.wait()

        for s in range(N_DEV - 1):
            slot = s % 2
            src = acc_ref if s == 0 else comm_ref.at[(s - 1) % 2]
            rdma = pltpu.make_async_remote_copy(
                src_ref=src,
                dst_ref=comm_ref.at[slot],
                send_sem=send_sems.at[3 + s],
                recv_sem=recv_sems.at[3 + s],
                device_id=(right,),
                device_id_type=pl.DeviceIdType.MESH,
            )
            rdma.start()
            rdma.wait()
            idx = jnp.mod(my - s, N_DEV)
            ocp = pltpu.make_async_copy(
                comm_ref.at[slot], out_ref.at[pl.ds(idx * CHUNK, CHUNK), :],
                copy_sems.at[3])
            ocp.start()
            ocp.wait()

        @functools.partial(
            pl.run_scoped, second_barrier=pltpu.SemaphoreType.REGULAR)
        def _(second_barrier):
            for nbr in (left, right):
                pl.semaphore_signal(
                    second_barrier, inc=1,
                    device_id=(nbr,), device_id_type=pl.DeviceIdType.MESH,
                )
            pl.semaphore_wait(second_barrier, 2)

    return pl.pallas_call(
        body,
        out_shape=jax.ShapeDtypeStruct((M, N), jnp.float32),
        in_specs=[pl.BlockSpec(memory_space=pl.ANY)],
        out_specs=pl.BlockSpec(memory_space=pl.ANY),
        scratch_shapes=[
            pltpu.VMEM((CHUNK, N), jnp.float32),
            pltpu.VMEM((CHUNK, N), jnp.float32),
            pltpu.VMEM((2, CHUNK, N), jnp.float32),
            pltpu.SemaphoreType.DMA((4,)),
            pltpu.SemaphoreType.DMA((6,)),
            pltpu.SemaphoreType.DMA((6,)),
        ],
        compiler_params=pltpu.CompilerParams(collective_id=0),
    )(partial)


# device time: 350135 ns/iter; 1.7801x vs baseline; 1.7801x over previous
import functools

import jax
import jax.numpy as jnp
from jax import lax
from jax.experimental import pallas as pl
from jax.experimental.pallas import tpu as pltpu

N_DEV = 4
M, N = 4096, 2048
CHUNK = M // N_DEV
HN = N // 2


def _silu(y):
    return y * (1.0 / (1.0 + jnp.exp(-y)))


def kernel(x, w_mat):
    partial = jnp.dot(x, w_mat, preferred_element_type=jnp.float32)

    def body(p_ref, out_ref, acc_ref, local_ref, comm_ref,
             copy_sems, send_sems, recv_sems):
        my = lax.axis_index("i")
        left = jnp.mod(my - 1, N_DEV)
        right = jnp.mod(my + 1, N_DEV)
        dsts = (right, left)

        barrier = pltpu.get_barrier_semaphore()
        for nbr in (left, right):
            pl.semaphore_signal(
                barrier, inc=1,
                device_id=(nbr,), device_id_type=pl.DeviceIdType.MESH,
            )
        pl.semaphore_wait(barrier, 2)

        def rs_send_idx(d, s):
            return jnp.mod(my - s, N_DEV) if d == 0 else jnp.mod(my + s, N_DEV)

        def rs_recv_idx(d, s):
            return (jnp.mod(my - s - 1, N_DEV) if d == 0
                    else jnp.mod(my + s + 1, N_DEV))

        def ag_recv_idx(d, s):
            return jnp.mod(my - s, N_DEV) if d == 0 else jnp.mod(my + s, N_DEV)

        for d in range(2):
            cp = pltpu.make_async_copy(
                p_ref.at[pl.ds(rs_send_idx(d, 0) * CHUNK, CHUNK),
                         pl.ds(d * HN, HN)],
                acc_ref.at[d], copy_sems.at[d])
            cp.start()
        for d in range(2):
            pltpu.make_async_copy(
                p_ref.at[pl.ds(rs_send_idx(d, 0) * CHUNK, CHUNK),
                         pl.ds(d * HN, HN)],
                acc_ref.at[d], copy_sems.at[d]).wait()

        for s in range(N_DEV - 1):
            slot = s % 2
            rdmas = []
            for d in range(2):
                rdma = pltpu.make_async_remote_copy(
                    src_ref=acc_ref.at[d],
                    dst_ref=comm_ref.at[d, slot],
                    send_sem=send_sems.at[d, s],
                    recv_sem=recv_sems.at[d, s],
                    device_id=(dsts[d],),
                    device_id_type=pl.DeviceIdType.MESH,
                )
                rdma.start()
                rdmas.append(rdma)
            lcps = []
            for d in range(2):
                lcp = pltpu.make_async_copy(
                    p_ref.at[pl.ds(rs_recv_idx(d, s) * CHUNK, CHUNK),
                             pl.ds(d * HN, HN)],
                    local_ref.at[d], copy_sems.at[d])
                lcp.start()
                lcps.append(lcp)
            for d in range(2):
                rdmas[d].wait()
                lcps[d].wait()
                acc_ref[d] = comm_ref[d, slot] + local_ref[d]

        ocps = []
        for d in range(2):
            acc_ref[d] = _silu(acc_ref[d])
            r = jnp.mod(my + 1, N_DEV) if d == 0 else jnp.mod(my - 1, N_DEV)
            ocp = pltpu.make_async_copy(
                acc_ref.at[d],
                out_ref.at[pl.ds(r * CHUNK, CHUNK), pl.ds(d * HN, HN)],
                copy_sems.at[2 + d])
            ocp.start()
            ocps.append(ocp)

        for s in range(N_DEV - 1):
            slot = s % 2
            rdmas = []
            for d in range(2):
                src = (acc_ref.at[d] if s == 0
                       else comm_ref.at[d, (s - 1) % 2])
                rdma = pltpu.make_async_remote_copy(
                    src_ref=src,
                    dst_ref=comm_ref.at[d, slot],
                    send_sem=send_sems.at[d, 3 + s],
                    recv_sem=recv_sems.at[d, 3 + s],
                    device_id=(dsts[d],),
                    device_id_type=pl.DeviceIdType.MESH,
                )
                rdma.start()
                rdmas.append(rdma)
            if s == 0:
                for ocp in ocps:
                    ocp.wait()
                ocps = []
            for d in range(2):
                rdmas[d].wait()
                ocp = pltpu.make_async_copy(
                    comm_ref.at[d, slot],
                    out_ref.at[pl.ds(ag_recv_idx(d, s) * CHUNK, CHUNK),
                               pl.ds(d * HN, HN)],
                    copy_sems.at[2 + d])
                ocp.start()
                ocps.append(ocp)
            for ocp in ocps:
                ocp.wait()
            ocps = []

        @functools.partial(
            pl.run_scoped, second_barrier=pltpu.SemaphoreType.REGULAR)
        def _(second_barrier):
            for nbr in (left, right):
                pl.semaphore_signal(
                    second_barrier, inc=1,
                    device_id=(nbr,), device_id_type=pl.DeviceIdType.MESH,
                )
            pl.semaphore_wait(second_barrier, 2)

    return pl.pallas_call(
        body,
        out_shape=jax.ShapeDtypeStruct((M, N), jnp.float32),
        in_specs=[pl.BlockSpec(memory_space=pl.ANY)],
        out_specs=pl.BlockSpec(memory_space=pl.ANY),
        scratch_shapes=[
            pltpu.VMEM((2, CHUNK, HN), jnp.float32),
            pltpu.VMEM((2, CHUNK, HN), jnp.float32),
            pltpu.VMEM((2, 2, CHUNK, HN), jnp.float32),
            pltpu.SemaphoreType.DMA((4,)),
            pltpu.SemaphoreType.DMA((2, 6)),
            pltpu.SemaphoreType.DMA((2, 6)),
        ],
        compiler_params=pltpu.CompilerParams(collective_id=0),
    )(partial)


# device time: 327762 ns/iter; 1.9016x vs baseline; 1.0683x over previous
import functools

import jax
import jax.numpy as jnp
from jax import lax
from jax.experimental import pallas as pl
from jax.experimental.pallas import tpu as pltpu

N_DEV = 4
M, K = 4096, 1024
N = 2048
CHUNK = M // N_DEV
HN = N // 2


def _silu(y):
    return y * (1.0 / (1.0 + jnp.exp(-y)))


def kernel(x, w_mat):
    def body(x_ref, w_ref, out_ref, acc_ref, local_ref, comm_ref,
             copy_sems, send_sems, recv_sems):
        my = lax.axis_index("i")
        left = jnp.mod(my - 1, N_DEV)
        right = jnp.mod(my + 1, N_DEV)
        dsts = (right, left)

        def gemm_chunk(d, idx, dst):
            dst[...] = jnp.dot(
                x_ref[pl.ds(idx * CHUNK, CHUNK), :],
                w_ref[:, pl.ds(d * HN, HN)],
                preferred_element_type=jnp.float32,
            )

        def rs_recv_idx(d, s):
            return (jnp.mod(my - s - 1, N_DEV) if d == 0
                    else jnp.mod(my + s + 1, N_DEV))

        def ag_recv_idx(d, s):
            return jnp.mod(my - s, N_DEV) if d == 0 else jnp.mod(my + s, N_DEV)

        for d in range(2):
            gemm_chunk(d, my, acc_ref.at[d])

        barrier = pltpu.get_barrier_semaphore()
        for nbr in (left, right):
            pl.semaphore_signal(
                barrier, inc=1,
                device_id=(nbr,), device_id_type=pl.DeviceIdType.MESH,
            )
        pl.semaphore_wait(barrier, 2)

        for s in range(N_DEV - 1):
            slot = s % 2
            rdmas = []
            for d in range(2):
                rdma = pltpu.make_async_remote_copy(
                    src_ref=acc_ref.at[d],
                    dst_ref=comm_ref.at[d, slot],
                    send_sem=send_sems.at[d, s],
                    recv_sem=recv_sems.at[d, s],
                    device_id=(dsts[d],),
                    device_id_type=pl.DeviceIdType.MESH,
                )
                rdma.start()
                rdmas.append(rdma)
            for d in range(2):
                gemm_chunk(d, rs_recv_idx(d, s), local_ref.at[d])
            for d in range(2):
                rdmas[d].wait()
                acc_ref[d] = comm_ref[d, slot] + local_ref[d]

        ocps = []
        for d in range(2):
            acc_ref[d] = _silu(acc_ref[d])
            r = jnp.mod(my + 1, N_DEV) if d == 0 else jnp.mod(my - 1, N_DEV)
            ocp = pltpu.make_async_copy(
                acc_ref.at[d],
                out_ref.at[pl.ds(r * CHUNK, CHUNK), pl.ds(d * HN, HN)],
                copy_sems.at[d])
            ocp.start()
            ocps.append(ocp)

        for s in range(N_DEV - 1):
            slot = s % 2
            rdmas = []
            for d in range(2):
                src = (acc_ref.at[d] if s == 0
                       else comm_ref.at[d, (s - 1) % 2])
                rdma = pltpu.make_async_remote_copy(
                    src_ref=src,
                    dst_ref=comm_ref.at[d, slot],
                    send_sem=send_sems.at[d, 3 + s],
                    recv_sem=recv_sems.at[d, 3 + s],
                    device_id=(dsts[d],),
                    device_id_type=pl.DeviceIdType.MESH,
                )
                rdma.start()
                rdmas.append(rdma)
            for ocp in ocps:
                ocp.wait()
            ocps = []
            for d in range(2):
                rdmas[d].wait()
                ocp = pltpu.make_async_copy(
                    comm_ref.at[d, slot],
                    out_ref.at[pl.ds(ag_recv_idx(d, s) * CHUNK, CHUNK),
                               pl.ds(d * HN, HN)],
                    copy_sems.at[d])
                ocp.start()
                ocps.append(ocp)
        for ocp in ocps:
            ocp.wait()

        @functools.partial(
            pl.run_scoped, second_barrier=pltpu.SemaphoreType.REGULAR)
        def _(second_barrier):
            for nbr in (left, right):
                pl.semaphore_signal(
                    second_barrier, inc=1,
                    device_id=(nbr,), device_id_type=pl.DeviceIdType.MESH,
                )
            pl.semaphore_wait(second_barrier, 2)

    return pl.pallas_call(
        body,
        out_shape=jax.ShapeDtypeStruct((M, N), jnp.float32),
        in_specs=[
            pl.BlockSpec(memory_space=pltpu.VMEM),
            pl.BlockSpec(memory_space=pltpu.VMEM),
        ],
        out_specs=pl.BlockSpec(memory_space=pl.ANY),
        scratch_shapes=[
            pltpu.VMEM((2, CHUNK, HN), jnp.float32),
            pltpu.VMEM((2, CHUNK, HN), jnp.float32),
            pltpu.VMEM((2, 2, CHUNK, HN), jnp.float32),
            pltpu.SemaphoreType.DMA((2,)),
            pltpu.SemaphoreType.DMA((2, 6)),
            pltpu.SemaphoreType.DMA((2, 6)),
        ],
        compiler_params=pltpu.CompilerParams(
            collective_id=0, vmem_limit_bytes=62 * 1024 * 1024),
    )(x, w_mat)


# device time: 172515 ns/iter; 3.6128x vs baseline; 1.8999x over previous
import functools

import jax
import jax.numpy as jnp
from jax import lax
from jax.experimental import pallas as pl
from jax.experimental.pallas import tpu as pltpu

N_DEV = 4
M, K = 4096, 1024
N = 2048
CHUNK = M // N_DEV
HN = N // 2
SUB = 4
SR = CHUNK // SUB
N_HOP = 2 * (N_DEV - 1)


def _silu(y):
    return y * (1.0 / (1.0 + jnp.exp(-y)))


def kernel(x, w_mat):
    def body(x_ref, w_ref, out_ref, acc_ref, acc_bf_ref, local_ref,
             comm_ref, own_sems, out_sems, send_sems, recv_sems,
             x_vmem_ref, x_sems):
        my = lax.axis_index("i")
        left = jnp.mod(my - 1, N_DEV)
        right = jnp.mod(my + 1, N_DEV)
        dsts = (right, left)

        cids = [my, jnp.mod(my - 1, N_DEV), jnp.mod(my + 1, N_DEV),
                jnp.mod(my + 2, N_DEV)]
        slot_map = {(0, 0): 1, (1, 0): 2, (0, 1): 3, (1, 1): 3,
                    (0, 2): 2, (1, 2): 1}
        wait_at = {0: (1, 2), 1: (3,), 2: ()}

        def xdma(o):
            return pltpu.make_async_copy(
                x_ref.at[pl.ds(cids[o] * CHUNK, CHUNK), :],
                x_vmem_ref.at[o], x_sems.at[o])

        def gemm_chunk(d, s):
            local_ref[d] = jnp.dot(
                x_vmem_ref[slot_map[(d, s)]],
                w_ref[:, d * HN:(d + 1) * HN],
                preferred_element_type=jnp.float32,
            )

        def gemm_own_sub(d, j):
            lo = j * SR
            acc_ref[d, lo:lo + SR, :] = jnp.dot(
                x_vmem_ref[0, lo:lo + SR, :],
                w_ref[:, d * HN:(d + 1) * HN],
                preferred_element_type=jnp.float32,
            )
            acc_bf_ref[d, lo:lo + SR, :] = (
                acc_ref[d, lo:lo + SR, :].astype(jnp.bfloat16))

        def rs_recv_idx(d, s):
            return (jnp.mod(my - s - 1, N_DEV) if d == 0
                    else jnp.mod(my + s + 1, N_DEV))

        def ag_recv_idx(d, s):
            return jnp.mod(my - s, N_DEV) if d == 0 else jnp.mod(my + s, N_DEV)

        def acc_bf_sub(d, j):
            return acc_bf_ref.at[d, j * SR:(j + 1) * SR, :]

        def comm_sub(d, h, j):
            return comm_ref.at[d, h % 2, j * SR:(j + 1) * SR, :]

        def issue_send(d, h, j, src):
            rdma = pltpu.make_async_remote_copy(
                src_ref=src,
                dst_ref=comm_sub(d, h, j),
                send_sem=send_sems.at[d, h, j],
                recv_sem=recv_sems.at[d, h, j],
                device_id=(dsts[d],),
                device_id_type=pl.DeviceIdType.MESH,
            )
            rdma.start()
            rdmas[d][h][j] = rdma
            return rdma

        rdmas = [[[None] * SUB for _ in range(N_HOP)] for _ in range(2)]

        own_dma = xdma(0)
        own_dma.start()
        for o in (1, 2, 3):
            xdma(o).start()
        own_dma.wait()
        for d in range(2):
            gemm_own_sub(d, 0)

        barrier = pltpu.get_barrier_semaphore()
        for nbr in (left, right):
            pl.semaphore_signal(
                barrier, inc=1,
                device_id=(nbr,), device_id_type=pl.DeviceIdType.MESH,
            )
        pl.semaphore_wait(barrier, 2)

        for d in range(2):
            issue_send(d, 0, 0, acc_bf_sub(d, 0))
        for j in range(1, SUB):
            for d in range(2):
                gemm_own_sub(d, j)
                issue_send(d, 0, j, acc_bf_sub(d, j))

        own_ocps = []
        for s in range(N_DEV - 1):
            for o in wait_at[s]:
                xdma(o).wait()
            for d in range(2):
                gemm_chunk(d, s)
            for j in range(SUB):
                for d in range(2):
                    rdmas[d][s][j].wait()
                    lo = j * SR
                    recv = comm_ref[d, s % 2, lo:lo + SR, :].astype(
                        jnp.float32)
                    part = local_ref[d, lo:lo + SR, :]
                    if s < N_DEV - 2:
                        acc = recv + part
                        acc_bf_ref[d, lo:lo + SR, :] = acc.astype(jnp.bfloat16)
                        issue_send(d, s + 1, j, acc_bf_sub(d, j))
                    else:
                        acc = _silu(recv + part)
                        acc_ref[d, lo:lo + SR, :] = acc
                        acc_bf_ref[d, lo:lo + SR, :] = acc.astype(jnp.bfloat16)
                        issue_send(d, 3, j, acc_bf_sub(d, j))
                        r = (jnp.mod(my + 1, N_DEV) if d == 0
                             else jnp.mod(my - 1, N_DEV))
                        ocp = pltpu.make_async_copy(
                            acc_ref.at[d, lo:lo + SR, :],
                            out_ref.at[pl.ds(r * CHUNK + lo, SR),
                                       d * HN:(d + 1) * HN],
                            own_sems.at[d, j])
                        ocp.start()
                        own_ocps.append(ocp)

        prev_ocps = []
        for s in range(N_DEV - 1):
            h = 3 + s
            for ocp in prev_ocps:
                ocp.wait()
            prev_ocps = []
            for j in range(SUB):
                for d in range(2):
                    rdmas[d][h][j].wait()
                    if h < N_HOP - 1:
                        issue_send(d, h + 1, j, comm_sub(d, h, j))
                    lo = j * SR
                    local_ref[d, lo:lo + SR, :] = comm_ref[
                        d, h % 2, lo:lo + SR, :].astype(jnp.float32)
                    ocp = pltpu.make_async_copy(
                        local_ref.at[d, lo:lo + SR, :],
                        out_ref.at[pl.ds(ag_recv_idx(d, s) * CHUNK + lo, SR),
                                   d * HN:(d + 1) * HN],
                        out_sems.at[d, s, j])
                    ocp.start()
                    prev_ocps.append(ocp)
        for ocp in prev_ocps + own_ocps:
            ocp.wait()

        @functools.partial(
            pl.run_scoped, second_barrier=pltpu.SemaphoreType.REGULAR)
        def _(second_barrier):
            for nbr in (left, right):
                pl.semaphore_signal(
                    second_barrier, inc=1,
                    device_id=(nbr,), device_id_type=pl.DeviceIdType.MESH,
                )
            pl.semaphore_wait(second_barrier, 2)

    return pl.pallas_call(
        body,
        out_shape=jax.ShapeDtypeStruct((M, N), jnp.float32),
        in_specs=[
            pl.BlockSpec(memory_space=pl.ANY),
            pl.BlockSpec(memory_space=pltpu.VMEM),
        ],
        out_specs=pl.BlockSpec(memory_space=pl.ANY),
        scratch_shapes=[
            pltpu.VMEM((2, CHUNK, HN), jnp.float32),
            pltpu.VMEM((2, CHUNK, HN), jnp.bfloat16),
            pltpu.VMEM((2, CHUNK, HN), jnp.float32),
            pltpu.VMEM((2, 2, CHUNK, HN), jnp.bfloat16),
            pltpu.SemaphoreType.DMA((2, SUB)),
            pltpu.SemaphoreType.DMA((2, N_DEV - 1, SUB)),
            pltpu.SemaphoreType.DMA((2, N_HOP, SUB)),
            pltpu.SemaphoreType.DMA((2, N_HOP, SUB)),
            pltpu.VMEM((N_DEV, CHUNK, K), jnp.float32),
            pltpu.SemaphoreType.DMA((N_DEV,)),
        ],
        compiler_params=pltpu.CompilerParams(
            collective_id=0, vmem_limit_bytes=62 * 1024 * 1024),
    )(x, w_mat)
